# baseline (speedup 1.0000x reference)
import functools

import jax
import jax.numpy as jnp
from jax import lax
from jax.experimental import pallas as pl
from jax.experimental.pallas import tpu as pltpu
from jax.experimental.pallas import tpu_sc as plsc

DIM = 64
L = 16
NC, NS = 2, 16
NW = NC * NS
NBUF = 2


def _rsqrt(x):
    i = lax.bitcast_convert_type(x, jnp.int32)
    i = jnp.int32(0x5F3759DF) - lax.shift_right_logical(i, 1)
    y = lax.bitcast_convert_type(i, jnp.float32)
    for _ in range(4):
        y = y * (1.5 - 0.5 * x * y * y)
    return y


def _inv_norm(ss):
    rs = _rsqrt(jnp.maximum(ss, 1e-30))
    n = ss * rs
    return 1.0 / jnp.maximum(n, 1e-12)


def _sqrt(x):
    xc = jnp.maximum(x, 0.0)
    return xc * _rsqrt(jnp.maximum(xc, 1e-30))


def _make_kernel(batch, chunk):
    ept = batch // NW
    nchunk = ept // chunk
    groups = chunk // L
    mesh = plsc.VectorSubcoreMesh(core_axis_name="c", subcore_axis_name="s")

    @functools.partial(
        pl.kernel,
        mesh=mesh,
        compiler_params=pltpu.CompilerParams(
            needs_layout_passes=False, use_tc_tiling_on_sc=False),
        out_type=jax.ShapeDtypeStruct((NW, L), jnp.float32),
        scratch_types=(
            [pltpu.VMEM((NBUF, 5, chunk), jnp.int32)]
            + [pltpu.VMEM((chunk, DIM), jnp.float32) for _ in range(NBUF * 5)]
            + [pltpu.VMEM((L,), jnp.float32),
               pltpu.SemaphoreType.DMA,
               pltpu.SemaphoreType.DMA]
        ),
    )
    def transe_kernel(ent_hbm, rel_hbm, eidx_hbm, ridx_hbm, out_hbm,
                      idx_v, *rest):
        rows_v = [rest[b * 5:(b + 1) * 5] for b in range(NBUF)]
        acc_v, sem0, sem1 = rest[NBUF * 5:]
        wid = lax.axis_index("s") * NC + lax.axis_index("c")
        sems = [sem0, sem1]
        handles = [[], []]

        def fire(b, c):
            base = wid * ept + c * chunk
            pltpu.sync_copy(eidx_hbm.at[:, pl.ds(base, chunk)], idx_v.at[b, 0:4])
            pltpu.sync_copy(ridx_hbm.at[pl.ds(base, chunk)], idx_v.at[b, 4])
            for j in range(4):
                handles[b].append(
                    pltpu.async_copy(ent_hbm.at[idx_v.at[b, j]], rows_v[b][j], sems[b]))
            handles[b].append(
                pltpu.async_copy(rel_hbm.at[idx_v.at[b, 4]], rows_v[b][4], sems[b]))

        def drain(b):
            for h in handles[b]:
                h.wait()
            handles[b].clear()

        def make_group_body(b):
            def group_body(g, acc):
                row0 = jnp.full((L,), g * L, jnp.int32) + lax.iota(jnp.int32, L)
                z = jnp.zeros((L,), jnp.float32)
                ss_hp = ss_tp = ss_hn = ss_tn = ss_r = z
                d_hp_r = d_hp_tp = d_r_tp = d_hn_r = d_hn_tn = d_r_tn = z
                for d in range(DIM):
                    col = jnp.full((L,), d, jnp.int32)
                    hp = plsc.load_gather(rows_v[b][0], [row0, col])
                    tp = plsc.load_gather(rows_v[b][1], [row0, col])
                    hn = plsc.load_gather(rows_v[b][2], [row0, col])
                    tn = plsc.load_gather(rows_v[b][3], [row0, col])
                    r = plsc.load_gather(rows_v[b][4], [row0, col])
                    ss_hp += hp * hp
                    ss_tp += tp * tp
                    ss_hn += hn * hn
                    ss_tn += tn * tn
                    ss_r += r * r
                    d_hp_r += hp * r
                    d_hp_tp += hp * tp
                    d_r_tp += r * tp
                    d_hn_r += hn * r
                    d_hn_tn += hn * tn
                    d_r_tn += r * tn
                ihp, itp = _inv_norm(ss_hp), _inv_norm(ss_tp)
                ihn, itn = _inv_norm(ss_hn), _inv_norm(ss_tn)
                ir = _inv_norm(ss_r)
                rr = ss_r * ir * ir
                e2p = (ss_hp * ihp * ihp + rr + ss_tp * itp * itp
                       + 2.0 * (d_hp_r * ihp * ir - d_hp_tp * ihp * itp - d_r_tp * ir * itp))
                e2n = (ss_hn * ihn * ihn + rr + ss_tn * itn * itn
                       + 2.0 * (d_hn_r * ihn * ir - d_hn_tn * ihn * itn - d_r_tn * ir * itn))
                loss = jnp.maximum(1.0 + _sqrt(e2p) - _sqrt(e2n), 0.0)
                return acc + loss
            return group_body

        acc = jnp.zeros((L,), jnp.float32)
        for c in range(min(NBUF, nchunk)):
            fire(c, c)
        for c in range(nchunk):
            b = c % NBUF
            drain(b)
            acc = lax.fori_loop(0, groups, make_group_body(b), acc)
            if c + NBUF < nchunk:
                fire(b, c + NBUF)

        acc_v[...] = acc
        pltpu.sync_copy(acc_v, out_hbm.at[wid])

    return transe_kernel


def kernel(ent_emb, rel_emb, pos_pairs, neg_pairs, rels):
    batch = pos_pairs.shape[0]
    eidx = jnp.stack([pos_pairs[:, 0], pos_pairs[:, 1],
                      neg_pairs[:, 0], neg_pairs[:, 1]], axis=0).astype(jnp.int32)
    ridx = rels[:, 0].astype(jnp.int32)
    partial = _make_kernel(batch, 128)(ent_emb, rel_emb, eidx, ridx)
    return jnp.sum(partial) / batch
